# Initial kernel scaffold; baseline (speedup 1.0000x reference)
import jax
import jax.numpy as jnp
from jax.experimental import pallas as pl


def kernel(input_, target):
    raise NotImplementedError("write your pallas kernel here")


# speedup vs baseline: 16.5812x; 16.5812x over previous
import functools

import jax
import jax.numpy as jnp
from jax import lax
from jax.experimental import pallas as pl
from jax.experimental.pallas import tpu as pltpu

_DELTA_VAR = 0.5
_DELTA_DIST = 1.5
_ALPHA = 1.0
_BETA = 1.0
_GAMMA = 0.001
_K = 32


def _pass_a(x_ref, ids_ref, sums_ref, counts_ref):
    n = pl.program_id(1)
    x = x_ref[0]
    ids = ids_ref[0]
    nb = x.shape[1]
    oh = (lax.broadcasted_iota(jnp.int32, (_K, nb), 0) == ids).astype(jnp.float32)
    s = lax.dot_general(oh, x, (((1,), (1,)), ((), ())),
                        preferred_element_type=jnp.float32)
    c = jnp.sum(oh, axis=1, keepdims=True)

    @pl.when(n == 0)
    def _():
        sums_ref[0] = s
        counts_ref[0] = c

    @pl.when(n != 0)
    def _():
        sums_ref[0] += s
        counts_ref[0] += c


def _pass_b(nch, nbat, x_ref, ids_ref, sums_ref, counts_ref, out_ref, hsum_ref, acc_ref):
    b = pl.program_id(0)
    n = pl.program_id(1)

    @pl.when((b == 0) & (n == 0))
    def _():
        acc_ref[...] = jnp.zeros_like(acc_ref)

    @pl.when(n == 0)
    def _():
        hsum_ref[...] = jnp.zeros_like(hsum_ref)

    x = x_ref[0]
    ids = ids_ref[0]
    nb = x.shape[1]
    counts = counts_ref[0]
    safe = jnp.maximum(counts, 1.0)
    means = sums_ref[0] / safe
    oh = (lax.broadcasted_iota(jnp.int32, (_K, nb), 0) == ids).astype(jnp.float32)
    mg = lax.dot_general(means, oh, (((0,), (0,)), ((), ())),
                         preferred_element_type=jnp.float32)
    diff = x - mg
    d2 = jnp.sum(diff * diff, axis=0, keepdims=True)
    d = jnp.sqrt(d2 + 1e-12)
    h = jnp.square(jnp.maximum(d - _DELTA_VAR, 0.0))
    hs = lax.dot_general(oh, h, (((1,), (1,)), ((), ())),
                         preferred_element_type=jnp.float32)
    hsum_ref[...] += hs

    @pl.when(n == nch - 1)
    def _():
        var_term = jnp.sum(hsum_ref[...] / safe) / _K
        mm = means
        mm2 = jnp.sum(mm * mm, axis=1, keepdims=True)
        gram = lax.dot_general(mm, mm, (((1,), (1,)), ((), ())),
                               preferred_element_type=jnp.float32)
        pd2 = jnp.maximum(mm2 + mm2.T - 2.0 * gram, 0.0)
        pd = jnp.sqrt(pd2 + 1e-12)
        rep = jnp.square(jnp.maximum(2.0 * _DELTA_DIST - pd, 0.0))
        eye = (lax.broadcasted_iota(jnp.int32, (_K, _K), 0)
               == lax.broadcasted_iota(jnp.int32, (_K, _K), 1))
        rep = jnp.where(eye, 0.0, rep)
        dist_term = jnp.sum(rep) / (_K * (_K - 1))
        reg_term = jnp.sum(jnp.sqrt(mm2 + 1e-12)) / _K
        loss_b = _ALPHA * var_term + _BETA * dist_term + _GAMMA * reg_term
        acc_ref[...] = acc_ref[...] + loss_b

        @pl.when(b == nbat - 1)
        def _():
            out_ref[...] = acc_ref[...] / float(nbat)


def kernel(input_, target):
    bsz, c, h, w = input_.shape
    n = h * w
    nb = 8192
    nch = n // nb
    x = input_.reshape(bsz, c, n)
    ids = target.reshape(bsz, 1, n).astype(jnp.int32)

    x_spec = pl.BlockSpec((1, c, nb), lambda b, i: (b, 0, i))
    ids_spec = pl.BlockSpec((1, 1, nb), lambda b, i: (b, 0, i))

    sums, counts = pl.pallas_call(
        _pass_a,
        grid=(bsz, nch),
        in_specs=[x_spec, ids_spec],
        out_specs=[
            pl.BlockSpec((1, _K, c), lambda b, i: (b, 0, 0)),
            pl.BlockSpec((1, _K, 1), lambda b, i: (b, 0, 0)),
        ],
        out_shape=[
            jax.ShapeDtypeStruct((bsz, _K, c), jnp.float32),
            jax.ShapeDtypeStruct((bsz, _K, 1), jnp.float32),
        ],
    )(x, ids)

    loss = pl.pallas_call(
        functools.partial(_pass_b, nch, bsz),
        grid=(bsz, nch),
        in_specs=[
            x_spec,
            ids_spec,
            pl.BlockSpec((1, _K, c), lambda b, i: (b, 0, 0)),
            pl.BlockSpec((1, _K, 1), lambda b, i: (b, 0, 0)),
        ],
        out_specs=pl.BlockSpec((1, 1), lambda b, i: (0, 0)),
        out_shape=jax.ShapeDtypeStruct((1, 1), jnp.float32),
        scratch_shapes=[
            pltpu.VMEM((_K, 1), jnp.float32),
            pltpu.VMEM((1, 1), jnp.float32),
        ],
    )(x, ids, sums, counts)
    return loss[0, 0]
